# Initial kernel scaffold; baseline (speedup 1.0000x reference)
import functools

import jax
import jax.numpy as jnp
from jax import lax
from jax.experimental import pallas as pl
from jax.experimental.pallas import tpu as pltpu
from jax.experimental.pallas import tpu_sc as plsc

BS, N, K, D = 2, 10000, 32, 128
H_MOD = 16
MOD_IN = K + 2 * D + 1
MOD_OUT = K + 1 + D


BN1 = 200


def _mod_body(heb_ref, h_ref, dl_ref, prim_ref, pm_ref,
              w1_ref, b1_ref, w2_ref, b2_ref,
              gate_ref, dec_ref, nprim_ref, rowabs_ref):
    w1 = w1_ref[...]
    w2 = w2_ref[...]
    b1 = b1_ref[...]
    b2 = b2_ref[...]
    for b in range(BS):
        inp = jnp.concatenate(
            [heb_ref[b], h_ref[b], dl_ref[b], prim_ref[b]], axis=-1)
        prod = inp[:, None, :] * w1
        hidden = jnp.tanh(prod.sum(axis=-1, keepdims=True) + b1)
        out = (hidden * w2).sum(axis=1) + b2
        gate_ref[b] = jax.nn.sigmoid(out[:, :K])
        dec_ref[b] = jax.nn.sigmoid(out[:, K:K + 1])
        nprim_ref[b] = out[:, K + 1:]
        rowabs_ref[b] = jnp.mean(jnp.abs(pm_ref[b]), axis=-1, keepdims=True)


def _modulator(heb, h, dl3, prim, pm, w1, b1r, w2, b2):
    nblk = N // BN1
    f32 = jnp.float32
    return pl.pallas_call(
        _mod_body,
        grid=(nblk,),
        in_specs=[
            pl.BlockSpec((BS, BN1, K), lambda i: (0, i, 0)),
            pl.BlockSpec((BS, BN1, D), lambda i: (0, i, 0)),
            pl.BlockSpec((BS, BN1, 1), lambda i: (0, i, 0)),
            pl.BlockSpec((BS, BN1, D), lambda i: (0, i, 0)),
            pl.BlockSpec((BS, BN1, D), lambda i: (0, i, 0)),
            pl.BlockSpec((BN1, H_MOD, MOD_IN), lambda i: (i, 0, 0)),
            pl.BlockSpec((BN1, H_MOD, 1), lambda i: (i, 0, 0)),
            pl.BlockSpec((BN1, H_MOD, MOD_OUT), lambda i: (i, 0, 0)),
            pl.BlockSpec((BN1, MOD_OUT), lambda i: (i, 0)),
        ],
        out_specs=[
            pl.BlockSpec((BS, BN1, K), lambda i: (0, i, 0)),
            pl.BlockSpec((BS, BN1, 1), lambda i: (0, i, 0)),
            pl.BlockSpec((BS, BN1, D), lambda i: (0, i, 0)),
            pl.BlockSpec((BS, BN1, 1), lambda i: (0, i, 0)),
        ],
        out_shape=[
            jax.ShapeDtypeStruct((BS, N, K), f32),
            jax.ShapeDtypeStruct((BS, N, 1), f32),
            jax.ShapeDtypeStruct((BS, N, D), f32),
            jax.ShapeDtypeStruct((BS, N, 1), f32),
        ],
    )(heb, h, dl3, prim, pm, w1, b1r, w2, b2)


NPU = 4
UNITS = BS * N // NPU
NWORK = 32
ITERS = -(-UNITS // NWORK)


def _sc_body(pm_hbm, idx_hbm, gate_hbm, rowabs_hbm, iv_hbm, ma_hbm,
             rowabs_v, idx_v, gate_v, rows_v, iv_v, ma_v, sem):
    wid = lax.axis_index("s") * 2 + lax.axis_index("c")
    pltpu.sync_copy(rowabs_hbm, rowabs_v)

    def unit(i, carry):
        u = wid + NWORK * i

        @pl.when(u < UNITS)
        def _():
            b = u // (N // NPU)
            nloc = (u - b * (N // NPU)) * NPU
            row0 = u * (NPU * K)
            pltpu.sync_copy(idx_hbm.at[pl.ds(row0, NPU * K)], idx_v)
            pltpu.async_copy(pm_hbm.at[idx_v], rows_v, sem).wait()
            pltpu.sync_copy(gate_hbm.at[pl.ds(row0, NPU * K)], gate_v)
            for j in range(NPU):
                def kstep(k, acc):
                    base = j * K + k
                    g = plsc.load_gather(
                        gate_v, [jnp.full((16,), base, jnp.int32)])
                    return tuple(
                        acc[l] + g * rows_v[base, pl.ds(l * 16, 16)]
                        for l in range(8))
                acc = lax.fori_loop(
                    0, K, kstep,
                    tuple(jnp.zeros((16,), jnp.float32) for _ in range(8)))
                for l in range(8):
                    iv_v[j, pl.ds(l * 16, 16)] = acc[l]
            for v in range(8):
                ids = idx_v[pl.ds(v * 16, 16)]
                ma_v[pl.ds(v * 16, 16)] = plsc.load_gather(rowabs_v, [ids])
            pltpu.sync_copy(iv_v, iv_hbm.at[pl.ds(b * N + nloc, NPU)])
            pltpu.sync_copy(ma_v, ma_hbm.at[pl.ds(row0, NPU * K)])
        return carry

    lax.fori_loop(0, ITERS, unit, None)


def _sc_gather(pm2, idx2, gate_flat, rowabs_flat):
    f32 = jnp.float32
    mesh = plsc.VectorSubcoreMesh(core_axis_name="c", subcore_axis_name="s")
    fn = functools.partial(
        pl.kernel,
        out_type=[
            jax.ShapeDtypeStruct((BS * N, D), f32),
            jax.ShapeDtypeStruct((BS * N * K,), f32),
        ],
        mesh=mesh,
        scratch_types=[
            pltpu.VMEM((BS * N,), f32),
            pltpu.VMEM((NPU * K,), jnp.int32),
            pltpu.VMEM((NPU * K,), f32),
            pltpu.VMEM((NPU * K, D), f32),
            pltpu.VMEM((NPU, D), f32),
            pltpu.VMEM((NPU * K,), f32),
            pltpu.SemaphoreType.DMA,
        ],
    )(_sc_body)
    return fn(pm2, idx2, gate_flat, rowabs_flat)


BN3 = 400


def _dense_body(iv_ref, prim_ref, h_ref, dec_ref, nid_ref, ma_ref, gate_ref,
                wi_ref, wp_ref, wn_ref, wd_ref, sb1_ref, sw2_ref, sb2_ref,
                mh_ref, mn_ref, mp_ref, mb1_ref, mw2_ref, mb2_ref,
                out_ref, heb_ref):
    def dot(a, b):
        return lax.dot_general(a, b, (((1,), (0,)), ((), ())),
                               preferred_element_type=jnp.float32)
    iv = iv_ref[0] * (K ** -0.5)
    prim = prim_ref[0]
    nid = nid_ref[...]
    d = dec_ref[0]
    hs = (dot(iv, wi_ref[...]) + dot(prim, wp_ref[...]) + dot(nid, wn_ref[...])
          + d * wd_ref[...] + sb1_ref[...])
    update = jnp.tanh(dot(jnp.tanh(hs), sw2_ref[...]) + sb2_ref[...])
    h_new = d * h_ref[0] + (1.0 - d) * update
    mh = jnp.tanh(dot(h_new, mh_ref[...]) + dot(nid, mn_ref[...])
                  + dot(prim, mp_ref[...]) + mb1_ref[...])
    new_msg = jnp.tanh(dot(mh, mw2_ref[...]) + mb2_ref[...])
    out_ref[0, :, :D] = h_new
    out_ref[0, :, D:] = new_msg
    heb_ref[0] = ma_ref[0] * gate_ref[0]


def _dense(iv, prim, h, dec, nid, ma, gate,
           wi, wp, wn, wd, sb1, sw2, sb2, mh, mn, mp, mb1, mw2, mb2):
    nblk = N // BN3
    f32 = jnp.float32
    full = lambda r, c: pl.BlockSpec((r, c), lambda b, i: (0, 0))
    return pl.pallas_call(
        _dense_body,
        grid=(BS, nblk),
        in_specs=[
            pl.BlockSpec((1, BN3, D), lambda b, i: (b, i, 0)),
            pl.BlockSpec((1, BN3, D), lambda b, i: (b, i, 0)),
            pl.BlockSpec((1, BN3, D), lambda b, i: (b, i, 0)),
            pl.BlockSpec((1, BN3, 1), lambda b, i: (b, i, 0)),
            pl.BlockSpec((BN3, D), lambda b, i: (i, 0)),
            pl.BlockSpec((1, BN3, K), lambda b, i: (b, i, 0)),
            pl.BlockSpec((1, BN3, K), lambda b, i: (b, i, 0)),
            full(D, D), full(D, D), full(D, D), full(1, D), full(1, D),
            full(D, D), full(1, D),
            full(D, D), full(D, D), full(D, D), full(1, D), full(D, D),
            full(1, D),
        ],
        out_specs=[
            pl.BlockSpec((1, BN3, 2 * D), lambda b, i: (b, i, 0)),
            pl.BlockSpec((1, BN3, K), lambda b, i: (b, i, 0)),
        ],
        out_shape=[
            jax.ShapeDtypeStruct((BS, N, 2 * D), f32),
            jax.ShapeDtypeStruct((BS, N, K), f32),
        ],
    )(iv, prim, h, dec, nid, ma, gate,
      wi, wp, wn, wd, sb1, sw2, sb2, mh, mn, mp, mb1, mw2, mb2)


@jax.jit
def kernel(h, prev_messages, hebbian_traces, decay_logit, primitives,
           conn_indices,
           mod_w1, mod_b1, mod_w2, mod_b2,
           state_w1, state_b1, state_w2, state_b2,
           msg_w1, msg_b1, msg_w2, msg_b2, neuron_id):
    f32 = jnp.float32
    dl3 = decay_logit[..., None]
    b1r = mod_b1[..., None]

    gate, dec, nprim, rowabs = _modulator(
        hebbian_traces, h, dl3, primitives, prev_messages,
        mod_w1, b1r, mod_w2, mod_b2)

    idx2 = (conn_indices[None, :, :]
            + (jnp.arange(BS, dtype=jnp.int32) * N)[:, None, None])
    idx2 = idx2.reshape(-1)
    pm2 = prev_messages.reshape(BS * N, D)
    iv, ma = _sc_gather(pm2, idx2, gate.reshape(-1), rowabs.reshape(-1))

    s1t = state_w1.T
    wi, wp, wn = s1t[:D], s1t[D:2 * D], s1t[2 * D:3 * D]
    wd = s1t[3 * D:]
    m1t = msg_w1.T
    mh, mn, mp = m1t[:D], m1t[D:2 * D], m1t[2 * D:]
    out, new_heb = _dense(
        iv.reshape(BS, N, D), nprim, h, dec, neuron_id,
        ma.reshape(BS, N, K), gate,
        wi, wp, wn, wd, state_b1[None].astype(f32), state_w2.T,
        state_b2[None], mh, mn, mp, msg_b1[None], msg_w2.T, msg_b2[None])
    return out, new_heb


# speedup vs baseline: 1.8928x; 1.8928x over previous
import functools

import jax
import jax.numpy as jnp
from jax import lax
from jax.experimental import pallas as pl
from jax.experimental.pallas import tpu as pltpu
from jax.experimental.pallas import tpu_sc as plsc

BS, N, K, D = 2, 10000, 32, 128
H_MOD = 16
MOD_IN = K + 2 * D + 1
MOD_OUT = K + 1 + D


BN1 = 200


def _mod_body(heb_ref, h_ref, dl_ref, prim_ref, pm_ref,
              w1_ref, b1_ref, w2_ref, b2_ref,
              gate_ref, dec_ref, nprim_ref, rowabs_ref):
    w1 = w1_ref[...]
    w2 = w2_ref[...]
    b1 = b1_ref[...]
    b2 = b2_ref[...]
    for b in range(BS):
        inp = jnp.concatenate(
            [heb_ref[b], h_ref[b], dl_ref[b], prim_ref[b]], axis=-1)
        prod = inp[:, None, :] * w1
        hidden = jnp.tanh(prod.sum(axis=-1, keepdims=True) + b1)
        out = (hidden * w2).sum(axis=1) + b2
        gate_ref[b] = jax.nn.sigmoid(out[:, :K])
        dec_ref[b] = jax.nn.sigmoid(out[:, K:K + 1])
        nprim_ref[b] = out[:, K + 1:]
        rowabs_ref[b] = jnp.mean(jnp.abs(pm_ref[b]), axis=-1, keepdims=True)


def _modulator(heb, h, dl3, prim, pm, w1, b1r, w2, b2):
    nblk = N // BN1
    f32 = jnp.float32
    return pl.pallas_call(
        _mod_body,
        grid=(nblk,),
        in_specs=[
            pl.BlockSpec((BS, BN1, K), lambda i: (0, i, 0)),
            pl.BlockSpec((BS, BN1, D), lambda i: (0, i, 0)),
            pl.BlockSpec((BS, BN1, 1), lambda i: (0, i, 0)),
            pl.BlockSpec((BS, BN1, D), lambda i: (0, i, 0)),
            pl.BlockSpec((BS, BN1, D), lambda i: (0, i, 0)),
            pl.BlockSpec((BN1, H_MOD, MOD_IN), lambda i: (i, 0, 0)),
            pl.BlockSpec((BN1, H_MOD, 1), lambda i: (i, 0, 0)),
            pl.BlockSpec((BN1, H_MOD, MOD_OUT), lambda i: (i, 0, 0)),
            pl.BlockSpec((BN1, MOD_OUT), lambda i: (i, 0)),
        ],
        out_specs=[
            pl.BlockSpec((BS, BN1, K), lambda i: (0, i, 0)),
            pl.BlockSpec((BS, BN1, 1), lambda i: (0, i, 0)),
            pl.BlockSpec((BS, BN1, D), lambda i: (0, i, 0)),
            pl.BlockSpec((BS, BN1, 1), lambda i: (0, i, 0)),
        ],
        out_shape=[
            jax.ShapeDtypeStruct((BS, N, K), f32),
            jax.ShapeDtypeStruct((BS, N, 1), f32),
            jax.ShapeDtypeStruct((BS, N, D), f32),
            jax.ShapeDtypeStruct((BS, N, 1), f32),
        ],
    )(heb, h, dl3, prim, pm, w1, b1r, w2, b2)


NPU = 4
UNITS = BS * N // NPU
NWORK = 32
ITERS = -(-UNITS // NWORK)


def _sc_body(pm_hbm, idx_hbm, gate_hbm, rowabs_hbm, iv_hbm, ma_hbm,
             rowabs_v, idx_v, gate_v, rows_v, iv_v, ma_v, sem):
    wid = lax.axis_index("s") * 2 + lax.axis_index("c")
    pltpu.sync_copy(rowabs_hbm, rowabs_v)

    def unit(i, carry):
        u = wid + NWORK * i

        @pl.when(u < UNITS)
        def _():
            b = u // (N // NPU)
            nloc = (u - b * (N // NPU)) * NPU
            row0 = u * (NPU * K)
            pltpu.sync_copy(idx_hbm.at[pl.ds(row0, NPU * K)], idx_v)
            pltpu.async_copy(pm_hbm.at[idx_v], rows_v, sem).wait()
            pltpu.sync_copy(gate_hbm.at[pl.ds(row0, NPU * K)], gate_v)
            for j in range(NPU):
                def kstep(k, acc):
                    base = j * K + k
                    g = plsc.load_gather(
                        gate_v, [jnp.full((16,), base, jnp.int32)])
                    return tuple(
                        acc[l] + g * rows_v[base, pl.ds(l * 16, 16)]
                        for l in range(8))
                acc = lax.fori_loop(
                    0, K, kstep,
                    tuple(jnp.zeros((16,), jnp.float32) for _ in range(8)))
                for l in range(8):
                    iv_v[j, pl.ds(l * 16, 16)] = acc[l]
            for v in range(8):
                ids = idx_v[pl.ds(v * 16, 16)]
                ma_v[pl.ds(v * 16, 16)] = plsc.load_gather(rowabs_v, [ids])
            pltpu.sync_copy(iv_v, iv_hbm.at[pl.ds(b * N + nloc, NPU)])
            pltpu.sync_copy(ma_v, ma_hbm.at[pl.ds(row0, NPU * K)])
        return carry

    lax.fori_loop(0, ITERS, unit, None)


def _sc_gather(pm2, idx2, gate_flat, rowabs_flat):
    f32 = jnp.float32
    mesh = plsc.VectorSubcoreMesh(
        core_axis_name="c", subcore_axis_name="s", num_cores=2,
        num_subcores=16)
    fn = functools.partial(
        pl.kernel,
        out_type=[
            jax.ShapeDtypeStruct((BS * N, D), f32),
            jax.ShapeDtypeStruct((BS * N * K,), f32),
        ],
        mesh=mesh,
        scratch_types=[
            pltpu.VMEM((BS * N,), f32),
            pltpu.VMEM((NPU * K,), jnp.int32),
            pltpu.VMEM((NPU * K,), f32),
            pltpu.VMEM((NPU * K, D), f32),
            pltpu.VMEM((NPU, D), f32),
            pltpu.VMEM((NPU * K,), f32),
            pltpu.SemaphoreType.DMA,
        ],
        compiler_params=pltpu.CompilerParams(needs_layout_passes=False),
    )(_sc_body)
    return fn(pm2, idx2, gate_flat, rowabs_flat)


BN3 = 400


def _dense_body(iv_ref, prim_ref, h_ref, dec_ref, nid_ref, ma_ref, gate_ref,
                wi_ref, wp_ref, wn_ref, wd_ref, sb1_ref, sw2_ref, sb2_ref,
                mh_ref, mn_ref, mp_ref, mb1_ref, mw2_ref, mb2_ref,
                out_ref, heb_ref):
    def dot(a, b):
        return lax.dot_general(a, b, (((1,), (0,)), ((), ())),
                               preferred_element_type=jnp.float32)
    iv = iv_ref[0] * (K ** -0.5)
    prim = prim_ref[0]
    nid = nid_ref[...]
    d = dec_ref[0]
    hs = (dot(iv, wi_ref[...]) + dot(prim, wp_ref[...]) + dot(nid, wn_ref[...])
          + d * wd_ref[...] + sb1_ref[...])
    update = jnp.tanh(dot(jnp.tanh(hs), sw2_ref[...]) + sb2_ref[...])
    h_new = d * h_ref[0] + (1.0 - d) * update
    mh = jnp.tanh(dot(h_new, mh_ref[...]) + dot(nid, mn_ref[...])
                  + dot(prim, mp_ref[...]) + mb1_ref[...])
    new_msg = jnp.tanh(dot(mh, mw2_ref[...]) + mb2_ref[...])
    out_ref[0, :, :D] = h_new
    out_ref[0, :, D:] = new_msg
    heb_ref[0] = ma_ref[0] * gate_ref[0]


def _dense(iv, prim, h, dec, nid, ma, gate,
           wi, wp, wn, wd, sb1, sw2, sb2, mh, mn, mp, mb1, mw2, mb2):
    nblk = N // BN3
    f32 = jnp.float32
    full = lambda r, c: pl.BlockSpec((r, c), lambda b, i: (0, 0))
    return pl.pallas_call(
        _dense_body,
        grid=(BS, nblk),
        in_specs=[
            pl.BlockSpec((1, BN3, D), lambda b, i: (b, i, 0)),
            pl.BlockSpec((1, BN3, D), lambda b, i: (b, i, 0)),
            pl.BlockSpec((1, BN3, D), lambda b, i: (b, i, 0)),
            pl.BlockSpec((1, BN3, 1), lambda b, i: (b, i, 0)),
            pl.BlockSpec((BN3, D), lambda b, i: (i, 0)),
            pl.BlockSpec((1, BN3, K), lambda b, i: (b, i, 0)),
            pl.BlockSpec((1, BN3, K), lambda b, i: (b, i, 0)),
            full(D, D), full(D, D), full(D, D), full(1, D), full(1, D),
            full(D, D), full(1, D),
            full(D, D), full(D, D), full(D, D), full(1, D), full(D, D),
            full(1, D),
        ],
        out_specs=[
            pl.BlockSpec((1, BN3, 2 * D), lambda b, i: (b, i, 0)),
            pl.BlockSpec((1, BN3, K), lambda b, i: (b, i, 0)),
        ],
        out_shape=[
            jax.ShapeDtypeStruct((BS, N, 2 * D), f32),
            jax.ShapeDtypeStruct((BS, N, K), f32),
        ],
    )(iv, prim, h, dec, nid, ma, gate,
      wi, wp, wn, wd, sb1, sw2, sb2, mh, mn, mp, mb1, mw2, mb2)


@jax.jit
def kernel(h, prev_messages, hebbian_traces, decay_logit, primitives,
           conn_indices,
           mod_w1, mod_b1, mod_w2, mod_b2,
           state_w1, state_b1, state_w2, state_b2,
           msg_w1, msg_b1, msg_w2, msg_b2, neuron_id):
    f32 = jnp.float32
    dl3 = decay_logit[..., None]
    b1r = mod_b1[..., None]

    gate, dec, nprim, rowabs = _modulator(
        hebbian_traces, h, dl3, primitives, prev_messages,
        mod_w1, b1r, mod_w2, mod_b2)

    idx2 = (conn_indices[None, :, :]
            + (jnp.arange(BS, dtype=jnp.int32) * N)[:, None, None])
    idx2 = idx2.reshape(-1)
    pm2 = prev_messages.reshape(BS * N, D)
    iv, ma = _sc_gather(pm2, idx2, gate.reshape(-1), rowabs.reshape(-1))

    s1t = state_w1.T
    wi, wp, wn = s1t[:D], s1t[D:2 * D], s1t[2 * D:3 * D]
    wd = s1t[3 * D:]
    m1t = msg_w1.T
    mh, mn, mp = m1t[:D], m1t[D:2 * D], m1t[2 * D:]
    out, new_heb = _dense(
        iv.reshape(BS, N, D), nprim, h, dec, neuron_id,
        ma.reshape(BS, N, K), gate,
        wi, wp, wn, wd, state_b1[None].astype(f32), state_w2.T,
        state_b2[None], mh, mn, mp, msg_b1[None], msg_w2.T, msg_b2[None])
    return out, new_heb


# speedup vs baseline: 3.0673x; 1.6205x over previous
import functools

import jax
import jax.numpy as jnp
from jax import lax
from jax.experimental import pallas as pl
from jax.experimental.pallas import tpu as pltpu
from jax.experimental.pallas import tpu_sc as plsc

BS, N, K, D = 2, 10000, 32, 128
H_MOD = 16
MOD_IN = K + 2 * D + 1
MOD_OUT = K + 1 + D


BN1 = 256
NBLK1 = -(-N // BN1)


def _mod_body(heb_ref, h_ref, dl_ref, prim_ref, pm_ref,
              w1_ref, b1_ref, w2_ref, b2_ref,
              gate_ref, dec_ref, nprim_ref, rowabs_ref):
    w1 = w1_ref[...]
    b1 = b1_ref[...]
    w2 = w2_ref[...]
    b2 = b2_ref[...]
    for b in range(BS):
        h_t = jnp.transpose(h_ref[b], (1, 0))
        prim_t = jnp.transpose(prim_ref[b], (1, 0))
        inp = jnp.concatenate(
            [heb_ref[b], h_t, dl_ref[b], prim_t], axis=0)
        hidden = jnp.tanh((inp[:, None, :] * w1).sum(axis=0) + b1)
        acc = b2
        for hh in range(H_MOD):
            acc = acc + hidden[hh][None, :] * w2[:, hh, :]
        acc_t = jnp.transpose(acc, (1, 0))
        gate_ref[b] = jax.nn.sigmoid(acc_t[:, :K])
        dec_ref[b] = jax.nn.sigmoid(acc_t[:, K:K + 1])
        nprim_ref[b] = acc_t[:, K + 1:]
        rowabs_ref[b] = jnp.mean(jnp.abs(pm_ref[b]), axis=-1, keepdims=True)


def _modulator(heb_t, h, dl_t, prim, pm, w1t, b1t, w2t, b2t):
    f32 = jnp.float32
    return pl.pallas_call(
        _mod_body,
        grid=(NBLK1,),
        in_specs=[
            pl.BlockSpec((BS, K, BN1), lambda i: (0, 0, i)),
            pl.BlockSpec((BS, BN1, D), lambda i: (0, i, 0)),
            pl.BlockSpec((BS, 1, BN1), lambda i: (0, 0, i)),
            pl.BlockSpec((BS, BN1, D), lambda i: (0, i, 0)),
            pl.BlockSpec((BS, BN1, D), lambda i: (0, i, 0)),
            pl.BlockSpec((MOD_IN, H_MOD, BN1), lambda i: (0, 0, i)),
            pl.BlockSpec((H_MOD, BN1), lambda i: (0, i)),
            pl.BlockSpec((MOD_OUT, H_MOD, BN1), lambda i: (0, 0, i)),
            pl.BlockSpec((MOD_OUT, BN1), lambda i: (0, i)),
        ],
        out_specs=[
            pl.BlockSpec((BS, BN1, K), lambda i: (0, i, 0)),
            pl.BlockSpec((BS, BN1, 1), lambda i: (0, i, 0)),
            pl.BlockSpec((BS, BN1, D), lambda i: (0, i, 0)),
            pl.BlockSpec((BS, BN1, 1), lambda i: (0, i, 0)),
        ],
        out_shape=[
            jax.ShapeDtypeStruct((BS, N, K), f32),
            jax.ShapeDtypeStruct((BS, N, 1), f32),
            jax.ShapeDtypeStruct((BS, N, D), f32),
            jax.ShapeDtypeStruct((BS, N, 1), f32),
        ],
    )(heb_t, h, dl_t, prim, pm, w1t, b1t, w2t, b2t)


NPU = 4
UNITS = BS * N // NPU
NWORK = 32
ITERS = -(-UNITS // NWORK)


def _sc_body(pm_hbm, idx_hbm, gate_hbm, rowabs_hbm, iv_hbm, ma_hbm,
             rowabs_v, idx_v, gate_v, rows_v, iv_v, ma_v, sem):
    wid = lax.axis_index("s") * 2 + lax.axis_index("c")
    pltpu.sync_copy(rowabs_hbm, rowabs_v)

    def unit(i, carry):
        u = wid + NWORK * i

        @pl.when(u < UNITS)
        def _():
            b = u // (N // NPU)
            nloc = (u - b * (N // NPU)) * NPU
            row0 = u * (NPU * K)
            pltpu.sync_copy(idx_hbm.at[pl.ds(row0, NPU * K)], idx_v)
            pltpu.async_copy(pm_hbm.at[idx_v], rows_v, sem).wait()
            pltpu.sync_copy(gate_hbm.at[pl.ds(row0, NPU * K)], gate_v)
            for j in range(NPU):
                def kstep(k, acc):
                    base = j * K + k
                    g = plsc.load_gather(
                        gate_v, [jnp.full((16,), base, jnp.int32)])
                    return tuple(
                        acc[l] + g * rows_v[base, pl.ds(l * 16, 16)]
                        for l in range(8))
                acc = lax.fori_loop(
                    0, K, kstep,
                    tuple(jnp.zeros((16,), jnp.float32) for _ in range(8)))
                for l in range(8):
                    iv_v[j, pl.ds(l * 16, 16)] = acc[l]
            for v in range(8):
                ids = idx_v[pl.ds(v * 16, 16)]
                ma_v[pl.ds(v * 16, 16)] = plsc.load_gather(rowabs_v, [ids])
            pltpu.sync_copy(iv_v, iv_hbm.at[pl.ds(b * N + nloc, NPU)])
            pltpu.sync_copy(ma_v, ma_hbm.at[pl.ds(row0, NPU * K)])
        return carry

    lax.fori_loop(0, ITERS, unit, None)


def _sc_gather(pm2, idx2, gate_flat, rowabs_flat):
    f32 = jnp.float32
    mesh = plsc.VectorSubcoreMesh(
        core_axis_name="c", subcore_axis_name="s", num_cores=2,
        num_subcores=16)
    fn = functools.partial(
        pl.kernel,
        out_type=[
            jax.ShapeDtypeStruct((BS * N, D), f32),
            jax.ShapeDtypeStruct((BS * N * K,), f32),
        ],
        mesh=mesh,
        scratch_types=[
            pltpu.VMEM((BS * N,), f32),
            pltpu.VMEM((NPU * K,), jnp.int32),
            pltpu.VMEM((NPU * K,), f32),
            pltpu.VMEM((NPU * K, D), f32),
            pltpu.VMEM((NPU, D), f32),
            pltpu.VMEM((NPU * K,), f32),
            pltpu.SemaphoreType.DMA,
        ],
        compiler_params=pltpu.CompilerParams(needs_layout_passes=False),
    )(_sc_body)
    return fn(pm2, idx2, gate_flat, rowabs_flat)


BN3 = 400


def _dense_body(iv_ref, prim_ref, h_ref, dec_ref, nid_ref, ma_ref, gate_ref,
                wi_ref, wp_ref, wn_ref, wd_ref, sb1_ref, sw2_ref, sb2_ref,
                mh_ref, mn_ref, mp_ref, mb1_ref, mw2_ref, mb2_ref,
                out_ref, heb_ref):
    def dot(a, b):
        return lax.dot_general(a, b, (((1,), (0,)), ((), ())),
                               preferred_element_type=jnp.float32)
    iv = iv_ref[0] * (K ** -0.5)
    prim = prim_ref[0]
    nid = nid_ref[...]
    d = dec_ref[0]
    hs = (dot(iv, wi_ref[...]) + dot(prim, wp_ref[...]) + dot(nid, wn_ref[...])
          + d * wd_ref[...] + sb1_ref[...])
    update = jnp.tanh(dot(jnp.tanh(hs), sw2_ref[...]) + sb2_ref[...])
    h_new = d * h_ref[0] + (1.0 - d) * update
    mh = jnp.tanh(dot(h_new, mh_ref[...]) + dot(nid, mn_ref[...])
                  + dot(prim, mp_ref[...]) + mb1_ref[...])
    new_msg = jnp.tanh(dot(mh, mw2_ref[...]) + mb2_ref[...])
    out_ref[0, :, :D] = h_new
    out_ref[0, :, D:] = new_msg
    heb_ref[0] = ma_ref[0] * gate_ref[0]


def _dense(iv, prim, h, dec, nid, ma, gate,
           wi, wp, wn, wd, sb1, sw2, sb2, mh, mn, mp, mb1, mw2, mb2):
    nblk = N // BN3
    f32 = jnp.float32
    full = lambda r, c: pl.BlockSpec((r, c), lambda b, i: (0, 0))
    return pl.pallas_call(
        _dense_body,
        grid=(BS, nblk),
        in_specs=[
            pl.BlockSpec((1, BN3, D), lambda b, i: (b, i, 0)),
            pl.BlockSpec((1, BN3, D), lambda b, i: (b, i, 0)),
            pl.BlockSpec((1, BN3, D), lambda b, i: (b, i, 0)),
            pl.BlockSpec((1, BN3, 1), lambda b, i: (b, i, 0)),
            pl.BlockSpec((BN3, D), lambda b, i: (i, 0)),
            pl.BlockSpec((1, BN3, K), lambda b, i: (b, i, 0)),
            pl.BlockSpec((1, BN3, K), lambda b, i: (b, i, 0)),
            full(D, D), full(D, D), full(D, D), full(1, D), full(1, D),
            full(D, D), full(1, D),
            full(D, D), full(D, D), full(D, D), full(1, D), full(D, D),
            full(1, D),
        ],
        out_specs=[
            pl.BlockSpec((1, BN3, 2 * D), lambda b, i: (b, i, 0)),
            pl.BlockSpec((1, BN3, K), lambda b, i: (b, i, 0)),
        ],
        out_shape=[
            jax.ShapeDtypeStruct((BS, N, 2 * D), f32),
            jax.ShapeDtypeStruct((BS, N, K), f32),
        ],
    )(iv, prim, h, dec, nid, ma, gate,
      wi, wp, wn, wd, sb1, sw2, sb2, mh, mn, mp, mb1, mw2, mb2)


@jax.jit
def kernel(h, prev_messages, hebbian_traces, decay_logit, primitives,
           conn_indices,
           mod_w1, mod_b1, mod_w2, mod_b2,
           state_w1, state_b1, state_w2, state_b2,
           msg_w1, msg_b1, msg_w2, msg_b2, neuron_id):
    f32 = jnp.float32
    heb_t = hebbian_traces.transpose(0, 2, 1)
    dl_t = decay_logit[:, None, :]
    w1t = mod_w1.transpose(2, 1, 0)
    b1t = mod_b1.transpose(1, 0)
    w2t = mod_w2.transpose(2, 1, 0)
    b2t = mod_b2.transpose(1, 0)

    gate, dec, nprim, rowabs = _modulator(
        heb_t, h, dl_t, primitives, prev_messages, w1t, b1t, w2t, b2t)

    idx2 = (conn_indices[None, :, :]
            + (jnp.arange(BS, dtype=jnp.int32) * N)[:, None, None])
    idx2 = idx2.reshape(-1)
    pm2 = prev_messages.reshape(BS * N, D)
    iv, ma = _sc_gather(pm2, idx2, gate.reshape(-1), rowabs.reshape(-1))

    s1t = state_w1.T
    wi, wp, wn = s1t[:D], s1t[D:2 * D], s1t[2 * D:3 * D]
    wd = s1t[3 * D:]
    m1t = msg_w1.T
    mh, mn, mp = m1t[:D], m1t[D:2 * D], m1t[2 * D:]
    out, new_heb = _dense(
        iv.reshape(BS, N, D), nprim, h, dec, neuron_id,
        ma.reshape(BS, N, K), gate,
        wi, wp, wn, wd, state_b1[None].astype(f32), state_w2.T,
        state_b2[None], mh, mn, mp, msg_b1[None], msg_w2.T, msg_b2[None])
    return out, new_heb


# speedup vs baseline: 5.1803x; 1.6889x over previous
import functools

import jax
import jax.numpy as jnp
from jax import lax
from jax.experimental import pallas as pl
from jax.experimental.pallas import tpu as pltpu
from jax.experimental.pallas import tpu_sc as plsc

BS, N, K, D = 2, 10000, 32, 128
H_MOD = 16
MOD_IN = K + 2 * D + 1
MOD_OUT = K + 1 + D


BN1 = 256
NBLK1 = -(-N // BN1)


def _mod_body(heb_ref, h_ref, dl_ref, prim_ref, pm_ref,
              w1_ref, b1_ref, w2_ref, b2_ref,
              gate_ref, dec_ref, nprim_ref, rowabs_ref):
    w1 = w1_ref[...]
    b1 = b1_ref[...]
    w2 = w2_ref[...]
    b2 = b2_ref[...]
    for b in range(BS):
        h_t = jnp.transpose(h_ref[b], (1, 0))
        prim_t = jnp.transpose(prim_ref[b], (1, 0))
        inp = jnp.concatenate(
            [heb_ref[b], h_t, dl_ref[b], prim_t], axis=0)
        hidden = jnp.tanh((inp[:, None, :] * w1).sum(axis=0) + b1)
        acc = b2
        for hh in range(H_MOD):
            acc = acc + hidden[hh][None, :] * w2[:, hh, :]
        acc_t = jnp.transpose(acc, (1, 0))
        gate_ref[b] = jax.nn.sigmoid(acc_t[:, :K])
        dec_ref[b] = jax.nn.sigmoid(acc_t[:, K:K + 1])
        nprim_ref[b] = acc_t[:, K + 1:]
        rowabs_ref[b] = jnp.mean(jnp.abs(pm_ref[b]), axis=-1, keepdims=True)


def _modulator(heb_t, h, dl_t, prim, pm, w1t, b1t, w2t, b2t):
    f32 = jnp.float32
    return pl.pallas_call(
        _mod_body,
        grid=(NBLK1,),
        in_specs=[
            pl.BlockSpec((BS, K, BN1), lambda i: (0, 0, i)),
            pl.BlockSpec((BS, BN1, D), lambda i: (0, i, 0)),
            pl.BlockSpec((BS, 1, BN1), lambda i: (0, 0, i)),
            pl.BlockSpec((BS, BN1, D), lambda i: (0, i, 0)),
            pl.BlockSpec((BS, BN1, D), lambda i: (0, i, 0)),
            pl.BlockSpec((MOD_IN, H_MOD, BN1), lambda i: (0, 0, i)),
            pl.BlockSpec((H_MOD, BN1), lambda i: (0, i)),
            pl.BlockSpec((MOD_OUT, H_MOD, BN1), lambda i: (0, 0, i)),
            pl.BlockSpec((MOD_OUT, BN1), lambda i: (0, i)),
        ],
        out_specs=[
            pl.BlockSpec((BS, BN1, K), lambda i: (0, i, 0)),
            pl.BlockSpec((BS, BN1, 1), lambda i: (0, i, 0)),
            pl.BlockSpec((BS, BN1, D), lambda i: (0, i, 0)),
            pl.BlockSpec((BS, BN1, 1), lambda i: (0, i, 0)),
        ],
        out_shape=[
            jax.ShapeDtypeStruct((BS, N, K), f32),
            jax.ShapeDtypeStruct((BS, N, 1), f32),
            jax.ShapeDtypeStruct((BS, N, D), f32),
            jax.ShapeDtypeStruct((BS, N, 1), f32),
        ],
    )(heb_t, h, dl_t, prim, pm, w1t, b1t, w2t, b2t)


GN = 128
SUB = 8
GPB = -(-N // GN)
NG = BS * GPB
NWORK = 32
ITERS = -(-NG // NWORK)


def _sc_body(pm_hbm, idx_hbm, gate_hbm, rowabs_hbm, iv_hbm, ma_hbm,
             rowabs_v, idx_g, gate_g, ma_g, iv_g, rows_a, rows_b,
             sem_a, sem_b):
    i32 = jnp.int32
    wid = lax.axis_index("s") * 2 + lax.axis_index("c")
    pltpu.sync_copy(rowabs_hbm, rowabs_v)
    last = GPB - 1
    szl = N - last * GN

    def fire(s, rbuf, sem):
        off = s * (SUB * K)
        pltpu.async_copy(pm_hbm.at[idx_g.at[pl.ds(off, 128)]],
                         rbuf.at[pl.ds(0, 128)], sem)
        pltpu.async_copy(pm_hbm.at[idx_g.at[pl.ds(off + 128, 128)]],
                         rbuf.at[pl.ds(128, 128)], sem)

    def drain(rbuf, sem):
        pltpu.make_async_copy(pm_hbm.at[pl.ds(0, SUB * K)], rbuf, sem).wait()

    def compute(s, rbuf):
        for j in range(SUB):
            def kstep(k, acc):
                g = plsc.load_gather(
                    gate_g, [s * (SUB * K) + j * K + k
                             + jnp.zeros((16,), i32)])
                base = j * K + k
                return tuple(
                    acc[l] + g * rbuf[base, pl.ds(l * 16, 16)]
                    for l in range(8))
            acc = lax.fori_loop(
                0, K, kstep,
                tuple(jnp.zeros((16,), jnp.float32) for _ in range(8)),
                unroll=4)
            for l in range(8):
                iv_g[s * SUB + j, pl.ds(l * 16, 16)] = acc[l]
        def mstep(v, carry):
            off = s * (SUB * K) + v * 16
            ids = idx_g[pl.ds(off, 16)]
            ma_g[pl.ds(off, 16)] = plsc.load_gather(rowabs_v, [ids])
            return carry
        lax.fori_loop(0, SUB * K // 16, mstep, None, unroll=4)

    def group(it, carry):
        g = wid + NWORK * it

        @pl.when(g < NG)
        def _():
            b = g // GPB
            gl = g - b * GPB
            n0 = gl * GN
            row0 = (b * N + n0) * K
            full = gl < last
            nsub = jnp.where(full, GN // SUB, szl // SUB)

            @pl.when(full)
            def _():
                pltpu.sync_copy(idx_hbm.at[pl.ds(row0, GN * K)], idx_g)
                pltpu.sync_copy(gate_hbm.at[pl.ds(row0, GN * K)], gate_g)

            @pl.when(jnp.logical_not(full))
            def _():
                pltpu.sync_copy(idx_hbm.at[pl.ds(row0, szl * K)],
                                idx_g.at[pl.ds(0, szl * K)])
                pltpu.sync_copy(gate_hbm.at[pl.ds(row0, szl * K)],
                                gate_g.at[pl.ds(0, szl * K)])

            fire(0, rows_a, sem_a)

            def pair(pp, carry):
                se = 2 * pp
                fire(se + 1, rows_b, sem_b)
                drain(rows_a, sem_a)
                compute(se, rows_a)

                @pl.when(se + 2 < nsub)
                def _():
                    fire(se + 2, rows_a, sem_a)
                drain(rows_b, sem_b)
                compute(se + 1, rows_b)
                return carry

            lax.fori_loop(0, nsub // 2, pair, None)

            @pl.when(full)
            def _():
                pltpu.sync_copy(iv_g, iv_hbm.at[pl.ds(b * N + n0, GN)])
                pltpu.sync_copy(ma_g, ma_hbm.at[pl.ds(row0, GN * K)])

            @pl.when(jnp.logical_not(full))
            def _():
                pltpu.sync_copy(iv_g.at[pl.ds(0, szl)],
                                iv_hbm.at[pl.ds(b * N + n0, szl)])
                pltpu.sync_copy(ma_g.at[pl.ds(0, szl * K)],
                                ma_hbm.at[pl.ds(row0, szl * K)])
        return carry

    lax.fori_loop(0, ITERS, group, None)


def _sc_gather(pm2, idx2, gate_flat, rowabs_flat):
    f32 = jnp.float32
    mesh = plsc.VectorSubcoreMesh(
        core_axis_name="c", subcore_axis_name="s", num_cores=2,
        num_subcores=16)
    fn = functools.partial(
        pl.kernel,
        out_type=[
            jax.ShapeDtypeStruct((BS * N, D), f32),
            jax.ShapeDtypeStruct((BS * N * K,), f32),
        ],
        mesh=mesh,
        scratch_types=[
            pltpu.VMEM((BS * N,), f32),
            pltpu.VMEM((GN * K,), jnp.int32),
            pltpu.VMEM((GN * K,), f32),
            pltpu.VMEM((GN * K,), f32),
            pltpu.VMEM((GN, D), f32),
            pltpu.VMEM((SUB * K, D), f32),
            pltpu.VMEM((SUB * K, D), f32),
            pltpu.SemaphoreType.DMA,
            pltpu.SemaphoreType.DMA,
        ],
        compiler_params=pltpu.CompilerParams(needs_layout_passes=False),
    )(_sc_body)
    return fn(pm2, idx2, gate_flat, rowabs_flat)


BN3 = 400


def _dense_body(iv_ref, prim_ref, h_ref, dec_ref, nid_ref, ma_ref, gate_ref,
                wi_ref, wp_ref, wn_ref, wd_ref, sb1_ref, sw2_ref, sb2_ref,
                mh_ref, mn_ref, mp_ref, mb1_ref, mw2_ref, mb2_ref,
                out_ref, heb_ref):
    def dot(a, b):
        return lax.dot_general(a, b, (((1,), (0,)), ((), ())),
                               preferred_element_type=jnp.float32)
    iv = iv_ref[0] * (K ** -0.5)
    prim = prim_ref[0]
    nid = nid_ref[...]
    d = dec_ref[0]
    hs = (dot(iv, wi_ref[...]) + dot(prim, wp_ref[...]) + dot(nid, wn_ref[...])
          + d * wd_ref[...] + sb1_ref[...])
    update = jnp.tanh(dot(jnp.tanh(hs), sw2_ref[...]) + sb2_ref[...])
    h_new = d * h_ref[0] + (1.0 - d) * update
    mh = jnp.tanh(dot(h_new, mh_ref[...]) + dot(nid, mn_ref[...])
                  + dot(prim, mp_ref[...]) + mb1_ref[...])
    new_msg = jnp.tanh(dot(mh, mw2_ref[...]) + mb2_ref[...])
    out_ref[0, :, :D] = h_new
    out_ref[0, :, D:] = new_msg
    heb_ref[0] = ma_ref[0] * gate_ref[0]


def _dense(iv, prim, h, dec, nid, ma, gate,
           wi, wp, wn, wd, sb1, sw2, sb2, mh, mn, mp, mb1, mw2, mb2):
    nblk = N // BN3
    f32 = jnp.float32
    full = lambda r, c: pl.BlockSpec((r, c), lambda b, i: (0, 0))
    return pl.pallas_call(
        _dense_body,
        grid=(BS, nblk),
        in_specs=[
            pl.BlockSpec((1, BN3, D), lambda b, i: (b, i, 0)),
            pl.BlockSpec((1, BN3, D), lambda b, i: (b, i, 0)),
            pl.BlockSpec((1, BN3, D), lambda b, i: (b, i, 0)),
            pl.BlockSpec((1, BN3, 1), lambda b, i: (b, i, 0)),
            pl.BlockSpec((BN3, D), lambda b, i: (i, 0)),
            pl.BlockSpec((1, BN3, K), lambda b, i: (b, i, 0)),
            pl.BlockSpec((1, BN3, K), lambda b, i: (b, i, 0)),
            full(D, D), full(D, D), full(D, D), full(1, D), full(1, D),
            full(D, D), full(1, D),
            full(D, D), full(D, D), full(D, D), full(1, D), full(D, D),
            full(1, D),
        ],
        out_specs=[
            pl.BlockSpec((1, BN3, 2 * D), lambda b, i: (b, i, 0)),
            pl.BlockSpec((1, BN3, K), lambda b, i: (b, i, 0)),
        ],
        out_shape=[
            jax.ShapeDtypeStruct((BS, N, 2 * D), f32),
            jax.ShapeDtypeStruct((BS, N, K), f32),
        ],
    )(iv, prim, h, dec, nid, ma, gate,
      wi, wp, wn, wd, sb1, sw2, sb2, mh, mn, mp, mb1, mw2, mb2)


@jax.jit
def kernel(h, prev_messages, hebbian_traces, decay_logit, primitives,
           conn_indices,
           mod_w1, mod_b1, mod_w2, mod_b2,
           state_w1, state_b1, state_w2, state_b2,
           msg_w1, msg_b1, msg_w2, msg_b2, neuron_id):
    f32 = jnp.float32
    heb_t = hebbian_traces.transpose(0, 2, 1)
    dl_t = decay_logit[:, None, :]
    w1t = mod_w1.transpose(2, 1, 0)
    b1t = mod_b1.transpose(1, 0)
    w2t = mod_w2.transpose(2, 1, 0)
    b2t = mod_b2.transpose(1, 0)

    gate, dec, nprim, rowabs = _modulator(
        heb_t, h, dl_t, primitives, prev_messages, w1t, b1t, w2t, b2t)

    idx2 = (conn_indices[None, :, :]
            + (jnp.arange(BS, dtype=jnp.int32) * N)[:, None, None])
    idx2 = idx2.reshape(-1)
    pm2 = prev_messages.reshape(BS * N, D)
    iv, ma = _sc_gather(pm2, idx2, gate.reshape(-1), rowabs.reshape(-1))

    s1t = state_w1.T
    wi, wp, wn = s1t[:D], s1t[D:2 * D], s1t[2 * D:3 * D]
    wd = s1t[3 * D:]
    m1t = msg_w1.T
    mh, mn, mp = m1t[:D], m1t[D:2 * D], m1t[2 * D:]
    out, new_heb = _dense(
        iv.reshape(BS, N, D), nprim, h, dec, neuron_id,
        ma.reshape(BS, N, K), gate,
        wi, wp, wn, wd, state_b1[None].astype(f32), state_w2.T,
        state_b2[None], mh, mn, mp, msg_b1[None], msg_w2.T, msg_b2[None])
    return out, new_heb
